# Initial kernel scaffold; baseline (speedup 1.0000x reference)
import jax
import jax.numpy as jnp
from jax.experimental import pallas as pl


def kernel(state, B0, B1):
    raise NotImplementedError("write your pallas kernel here")


# speedup vs baseline: 5.4609x; 5.4609x over previous
import jax
import jax.numpy as jnp
from jax.experimental import pallas as pl

U0 = 10.0
R = 0.2

_BN = 2048


def _ped_kernel(x_ref, y_ref, b0x_ref, b0y_ref, b1x_ref, b1y_ref, out_ref):
    x = x_ref[...]
    y = y_ref[...]

    def min_dist(bx_ref, by_ref):
        dx = x - bx_ref[...]
        dy = y - by_ref[...]
        d2 = dx * dx + dy * dy
        return jnp.min(d2, axis=1, keepdims=True)

    m0 = min_dist(b0x_ref, b0y_ref)
    m1 = min_dist(b1x_ref, b1y_ref)
    m = jnp.concatenate([m0, m1], axis=1)
    out_ref[...] = U0 * jnp.exp(-jnp.sqrt(m) / R)


def kernel(state, B0, B1):
    n = state.shape[0]
    x = state[:, 0:1]
    y = state[:, 1:2]
    b0x = B0[:, 0][None, :]
    b0y = B0[:, 1][None, :]
    b1x = B1[:, 0][None, :]
    b1y = B1[:, 1][None, :]

    grid = (n // _BN,)
    row_spec = pl.BlockSpec((_BN, 1), lambda i: (i, 0))
    b_spec = pl.BlockSpec((1, 64), lambda i: (0, 0))
    return pl.pallas_call(
        _ped_kernel,
        grid=grid,
        in_specs=[row_spec, row_spec, b_spec, b_spec, b_spec, b_spec],
        out_specs=pl.BlockSpec((_BN, 2), lambda i: (i, 0)),
        out_shape=jax.ShapeDtypeStruct((n, 2), jnp.float32),
    )(x, y, b0x, b0y, b1x, b1y)


# speedup vs baseline: 35.3600x; 6.4751x over previous
import jax
import jax.numpy as jnp
from jax.experimental import pallas as pl
from jax.experimental.pallas import tpu as pltpu

U0 = 10.0
R = 0.2

_ROWS = 512
_BR = 128
_M = 64


def _ped_kernel(b0_ref, b1_ref, x_ref, y_ref, o0_ref, o1_ref):
    x = x_ref[...]
    y = y_ref[...]

    def min_d2(b_ref):
        m = None
        for j in range(_M):
            dx = x - b_ref[j, 0]
            dy = y - b_ref[j, 1]
            d2 = dx * dx + dy * dy
            m = d2 if m is None else jnp.minimum(m, d2)
        return m

    o0_ref[...] = U0 * jnp.exp(-jnp.sqrt(min_d2(b0_ref)) / R)
    o1_ref[...] = U0 * jnp.exp(-jnp.sqrt(min_d2(b1_ref)) / R)


def kernel(state, B0, B1):
    xs = state[:, 0].reshape(_ROWS, 128)
    ys = state[:, 1].reshape(_ROWS, 128)

    grid = (_ROWS // _BR,)
    plane = pl.BlockSpec((_BR, 128), lambda i: (i, 0))
    smem = pl.BlockSpec(memory_space=pltpu.SMEM)
    o0, o1 = pl.pallas_call(
        _ped_kernel,
        grid=grid,
        in_specs=[smem, smem, plane, plane],
        out_specs=[plane, plane],
        out_shape=[jax.ShapeDtypeStruct((_ROWS, 128), jnp.float32)] * 2,
    )(B0, B1, xs, ys)
    return jnp.stack([o0.reshape(-1), o1.reshape(-1)], axis=1)
